# baseline (speedup 1.0000x reference)
import functools

import jax
import jax.numpy as jnp
from jax import lax
from jax.experimental import pallas as pl
from jax.experimental.pallas import tpu as pltpu
from jax.experimental.pallas import tpu_sc as plsc

N = 10000
NPAD = 10016
DISCARD = 10008
IN_CH = 128
HID = 16
OUT_CH = 40
E = 320000
NC = 1
NS = 16
NW = NC * NS
CHUNK = 128
NCHUNK = 2560
CPT = NCHUNK // NW
EPAD = NCHUNK * CHUNK

_MESH = plsc.VectorSubcoreMesh(
    core_axis_name="c", subcore_axis_name="s", num_cores=NC, num_subcores=NS
)


def _wid():
    return lax.axis_index("s") * NC + lax.axis_index("c")


def _load_chunks(ehbm, ev, w):
    start = pl.multiple_of(CPT * w, 8)
    pltpu.sync_copy(ehbm.at[pl.ds(start, CPT)], ev)


@functools.partial(
    pl.kernel,
    out_type=jax.ShapeDtypeStruct((NC, NPAD), jnp.float32),
    mesh=_MESH,
    scratch_types=[
        pltpu.VMEM((CPT, CHUNK), jnp.int32),
        pltpu.VMEM((CHUNK,), jnp.float32),
        pltpu.VMEM_SHARED((NPAD,), jnp.float32),
    ],
    compiler_params=pltpu.CompilerParams(use_tc_tiling_on_sc=False),
)
def _sc_degree(dst_hbm, zeros_hbm, out_hbm, dst_v, ones_v, acc):
    c = lax.axis_index("c")
    s = lax.axis_index("s")
    _load_chunks(dst_hbm, dst_v, _wid())
    for i in range(CHUNK // 16):
        ones_v[pl.ds(i * 16, 16)] = jnp.ones((16,), jnp.float32)

    @pl.when(s == 0)
    def _():
        pltpu.sync_copy(zeros_hbm, acc)

    plsc.subcore_barrier()

    def chunk(j, carry):
        pltpu.sync_copy(ones_v, acc.at[dst_v.at[j]], add=True)
        return carry

    lax.fori_loop(0, CPT, chunk, 0)
    plsc.subcore_barrier()

    @pl.when(s == 0)
    def _():
        pltpu.sync_copy(acc, out_hbm.at[c])


@functools.partial(
    pl.kernel,
    out_type=jax.ShapeDtypeStruct((NC, NPAD, HID), jnp.float32),
    mesh=_MESH,
    scratch_types=[
        pltpu.VMEM((CPT, CHUNK), jnp.int32),
        pltpu.VMEM((CPT, CHUNK), jnp.int32),
        pltpu.VMEM((CHUNK, HID), jnp.float32),
        pltpu.VMEM((CHUNK, HID), jnp.float32),
        pltpu.VMEM_SHARED((NPAD, HID), jnp.float32),
        pltpu.SemaphoreType.DMA,
        pltpu.SemaphoreType.DMA,
    ],
    compiler_params=pltpu.CompilerParams(use_tc_tiling_on_sc=False),
)
def _sc_propagate(table_hbm, src_hbm, dst_hbm, zeros_hbm, out_hbm,
                  src_v, dst_v, rows_a, rows_b, acc, sem_a, sem_b):
    c = lax.axis_index("c")
    s = lax.axis_index("s")
    w = _wid()
    _load_chunks(src_hbm, src_v, w)
    _load_chunks(dst_hbm, dst_v, w)

    @pl.when(s == 0)
    def _():
        pltpu.sync_copy(zeros_hbm, acc)

    plsc.subcore_barrier()

    def gather(k, buf, sem):
        pltpu.async_copy(table_hbm.at[src_v.at[k]], buf, sem)

    def drain(buf, sem):
        pltpu.make_async_copy(table_hbm.at[src_v.at[0]], buf, sem).wait()

    def scatter(k, buf):
        pltpu.sync_copy(buf, acc.at[dst_v.at[k]], add=True)

    gather(0, rows_a, sem_a)

    def pair(j, carry):
        k = 2 * j
        gather(k + 1, rows_b, sem_b)
        drain(rows_a, sem_a)
        scatter(k, rows_a)
        gather(k + 2, rows_a, sem_a)
        drain(rows_b, sem_b)
        scatter(k + 1, rows_b)
        return carry

    lax.fori_loop(0, CPT // 2 - 1, pair, 0)
    gather(CPT - 1, rows_b, sem_b)
    drain(rows_a, sem_a)
    scatter(CPT - 2, rows_a)
    drain(rows_b, sem_b)
    scatter(CPT - 1, rows_b)
    plsc.subcore_barrier()

    @pl.when(s == 0)
    def _():
        pltpu.sync_copy(acc, out_hbm.at[c])


def _tc1_body(x_ref, w1_ref, degp_ref, xs1_ref, dis_ref):
    deg = sum(degp_ref[i] for i in range(NC)) + 1.0
    dis = lax.rsqrt(deg)
    dis_ref[...] = dis[:N]
    xw = jnp.dot(x_ref[...], w1_ref[...], preferred_element_type=jnp.float32)
    xs1_ref[...] = xw * dis[:N]


def _tc2_body(y1p_ref, xs1_ref, dis_ref, b1_ref, xs2_ref):
    y = sum(y1p_ref[i, :N] for i in range(NC)) + xs1_ref[...]
    h = jnp.maximum(dis_ref[...] * y + b1_ref[...], 0.0)
    xs2_ref[...] = h * dis_ref[...]


def _tc3_body(y2p_ref, xs2_ref, dis_ref, w2_ref, b2_ref, out_ref):
    g = dis_ref[...] * (sum(y2p_ref[i, :N] for i in range(NC)) + xs2_ref[...])
    o = jnp.dot(g, w2_ref[...], preferred_element_type=jnp.float32) + b2_ref[...]
    m = jnp.max(o, axis=1, keepdims=True)
    e = o - m
    lse = jnp.log(jnp.sum(jnp.exp(e), axis=1, keepdims=True))
    out_ref[...] = e - lse


def kernel(x, edge_index, W1, b1, W2, b2):
    src = jnp.concatenate(
        [edge_index[0].astype(jnp.int32), jnp.zeros((EPAD - E,), jnp.int32)]
    ).reshape(NCHUNK, CHUNK)
    dst = jnp.concatenate(
        [edge_index[1].astype(jnp.int32),
         jnp.full((EPAD - E,), DISCARD, jnp.int32)]
    ).reshape(NCHUNK, CHUNK)
    z1 = jnp.zeros((NPAD,), jnp.float32)
    z16 = jnp.zeros((NPAD, HID), jnp.float32)

    degp = _sc_degree(dst, z1)

    xs1, dis = pl.pallas_call(
        _tc1_body,
        out_shape=(
            jax.ShapeDtypeStruct((N, HID), jnp.float32),
            jax.ShapeDtypeStruct((N, 1), jnp.float32),
        ),
    )(x, W1, degp.reshape(NC, NPAD, 1))

    y1p = _sc_propagate(xs1, src, dst, z16)

    xs2 = pl.pallas_call(
        _tc2_body,
        out_shape=jax.ShapeDtypeStruct((N, HID), jnp.float32),
    )(y1p, xs1, dis, b1.reshape(1, HID))

    y2p = _sc_propagate(xs2, src, dst, z16)

    out = pl.pallas_call(
        _tc3_body,
        out_shape=jax.ShapeDtypeStruct((N, OUT_CH), jnp.float32),
    )(y2p, xs2, dis, W2, b2.reshape(1, OUT_CH))
    return out


# speedup vs baseline: 1.4263x; 1.4263x over previous
import functools

import jax
import jax.numpy as jnp
from jax import lax
from jax.experimental import pallas as pl
from jax.experimental.pallas import tpu as pltpu
from jax.experimental.pallas import tpu_sc as plsc

N = 10000
NPAD = 10016
DISCARD = 10008
IN_CH = 128
HID = 16
OUT_CH = 40
E = 320000
NC = 2
NS = 16
NW = NC * NS
CHUNK = 128
NCHUNK = 2560
CPT = NCHUNK // NW
EPAD = NCHUNK * CHUNK
NBUF = 8
NSUP = CPT // NBUF

_MESH = plsc.VectorSubcoreMesh(
    core_axis_name="c", subcore_axis_name="s", num_cores=NC, num_subcores=NS
)
_SC_PARAMS = pltpu.CompilerParams(use_tc_tiling_on_sc=False)


def _wid():
    return lax.axis_index("s") * NC + lax.axis_index("c")


def _load_chunks(ehbm, which, ev, w):
    start = pl.multiple_of(CPT * w, 8)
    pltpu.sync_copy(ehbm.at[which, pl.ds(start, CPT)], ev)


@functools.partial(
    pl.kernel,
    out_type=jax.ShapeDtypeStruct((NC, NPAD), jnp.float32),
    mesh=_MESH,
    scratch_types=[
        pltpu.VMEM((CPT, CHUNK), jnp.int32),
        pltpu.VMEM((CHUNK,), jnp.float32),
        pltpu.VMEM_SHARED((NPAD,), jnp.float32),
        pltpu.SemaphoreType.DMA,
    ],
    compiler_params=_SC_PARAMS,
)
def _sc_degree(edges_hbm, zeros_hbm, out_hbm, dst_v, ones_v, acc, sem):
    c = lax.axis_index("c")
    s = lax.axis_index("s")
    _load_chunks(edges_hbm, 1, dst_v, _wid())
    for i in range(CHUNK // 16):
        ones_v[pl.ds(i * 16, 16)] = jnp.ones((16,), jnp.float32)

    @pl.when(s == 0)
    def _():
        pltpu.sync_copy(zeros_hbm, acc)

    plsc.subcore_barrier()

    def fire(j, carry):
        pltpu.async_copy(ones_v, acc.at[dst_v.at[j]], sem, add=True)
        return carry

    lax.fori_loop(0, CPT, fire, 0)

    def drain(j, carry):
        pltpu.make_async_copy(ones_v, acc.at[dst_v.at[0]], sem).wait()
        return carry

    lax.fori_loop(0, CPT, drain, 0)
    plsc.subcore_barrier()

    @pl.when(s == 0)
    def _():
        pltpu.sync_copy(acc, out_hbm.at[c])


@functools.partial(
    pl.kernel,
    out_type=jax.ShapeDtypeStruct((NC, NPAD, HID), jnp.float32),
    mesh=_MESH,
    scratch_types=[
        pltpu.VMEM((CPT, CHUNK), jnp.int32),
        pltpu.VMEM((CPT, CHUNK), jnp.int32),
        [pltpu.VMEM((CHUNK, HID), jnp.float32)] * NBUF,
        [pltpu.SemaphoreType.DMA] * NBUF,
        [pltpu.SemaphoreType.DMA] * NBUF,
        pltpu.VMEM_SHARED((NPAD, HID), jnp.float32),
    ],
    compiler_params=_SC_PARAMS,
)
def _sc_propagate(table_hbm, edges_hbm, zeros_hbm, out_hbm,
                  src_v, dst_v, rows, gsem, ssem, acc):
    c = lax.axis_index("c")
    s = lax.axis_index("s")
    w = _wid()
    _load_chunks(edges_hbm, 0, src_v, w)
    _load_chunks(edges_hbm, 1, dst_v, w)

    @pl.when(s == 0)
    def _():
        pltpu.sync_copy(zeros_hbm, acc)

    plsc.subcore_barrier()

    def gather(k, i):
        pltpu.async_copy(table_hbm.at[src_v.at[k]], rows[i], gsem[i])

    def gather_done(i):
        pltpu.make_async_copy(table_hbm.at[src_v.at[0]], rows[i],
                              gsem[i]).wait()

    def scatter(k, i):
        pltpu.async_copy(rows[i], acc.at[dst_v.at[k]], ssem[i], add=True)

    def scatter_done(i):
        pltpu.make_async_copy(rows[i], acc.at[dst_v.at[0]], ssem[i]).wait()

    for i in range(NBUF):
        gather(i, i)

    def superchunk(t, carry):
        base = NBUF * t
        for i in range(NBUF):
            k = base + i
            gather_done(i)
            scatter(k, i)

            @pl.when(t < NSUP - 1)
            def _():
                scatter_done(i)
                gather(k + NBUF, i)
        return carry

    lax.fori_loop(0, NSUP, superchunk, 0)
    for i in range(NBUF):
        scatter_done(i)
    plsc.subcore_barrier()

    @pl.when(s == 0)
    def _():
        pltpu.sync_copy(acc, out_hbm.at[c])


def _tc_mm_body(x_ref, w1_ref, xw_ref):
    xw_ref[...] = jnp.dot(x_ref[...], w1_ref[...],
                          preferred_element_type=jnp.float32)


def _tc1_body(xw_ref, degp_ref, xs1_ref, dis_ref):
    deg = sum(degp_ref[i] for i in range(NC)) + 1.0
    dis = lax.rsqrt(deg)
    dis_ref[...] = dis
    xs1_ref[...] = xw_ref[...] * dis


def _tc2_body(y1p_ref, xs1_ref, dis_ref, b1_ref, xs2_ref):
    y = sum(y1p_ref[i, :N] for i in range(NC)) + xs1_ref[...]
    h = jnp.maximum(dis_ref[...] * y + b1_ref[...], 0.0)
    xs2_ref[...] = h * dis_ref[...]


def _tc3_body(y2p_ref, xs2_ref, dis_ref, w2_ref, b2_ref, out_ref):
    g = dis_ref[...] * (sum(y2p_ref[i, :N] for i in range(NC)) + xs2_ref[...])
    o = jnp.dot(g, w2_ref[...], preferred_element_type=jnp.float32) + b2_ref[...]
    m = jnp.max(o, axis=1, keepdims=True)
    e = o - m
    lse = jnp.log(jnp.sum(jnp.exp(e), axis=1, keepdims=True))
    out_ref[...] = e - lse


def kernel(x, edge_index, W1, b1, W2, b2):
    e32 = edge_index.astype(jnp.int32)
    padcols = jnp.stack([
        jnp.zeros((EPAD - E,), jnp.int32),
        jnp.full((EPAD - E,), DISCARD, jnp.int32),
    ])
    edges = jnp.concatenate([e32, padcols], axis=1).reshape(2, NCHUNK, CHUNK)
    z1 = jnp.zeros((NPAD,), jnp.float32)
    z16 = jnp.zeros((NPAD, HID), jnp.float32)

    degp = _sc_degree(edges, z1)
    xw = pl.pallas_call(
        _tc_mm_body,
        out_shape=jax.ShapeDtypeStruct((N, HID), jnp.float32),
    )(x, W1)

    xs1, dis = pl.pallas_call(
        _tc1_body,
        out_shape=(
            jax.ShapeDtypeStruct((N, HID), jnp.float32),
            jax.ShapeDtypeStruct((N, 1), jnp.float32),
        ),
    )(xw, degp[:, :N].reshape(NC, N, 1))

    y1p = _sc_propagate(xs1, edges, z16)

    xs2 = pl.pallas_call(
        _tc2_body,
        out_shape=jax.ShapeDtypeStruct((N, HID), jnp.float32),
    )(y1p, xs1, dis, b1.reshape(1, HID))

    y2p = _sc_propagate(xs2, edges, z16)

    out = pl.pallas_call(
        _tc3_body,
        out_shape=jax.ShapeDtypeStruct((N, OUT_CH), jnp.float32),
    )(y2p, xs2, dis, W2, b2.reshape(1, OUT_CH))
    return out


# speedup vs baseline: 1.8947x; 1.3284x over previous
import functools

import jax
import jax.numpy as jnp
from jax import lax
from jax.experimental import pallas as pl
from jax.experimental.pallas import tpu as pltpu
from jax.experimental.pallas import tpu_sc as plsc

N = 10000
NPAD = 10016
DISCARD = 10008
IN_CH = 128
HID = 16
OUT_CH = 40
E = 320000
NC = 2
NS = 16
NW = NC * NS
CHUNK = 128
NCHUNK = 2560
CPT = NCHUNK // NW
EPAD = NCHUNK * CHUNK
NBUF = 8
NSUP = CPT // NBUF
ROWS_PT = 632
ROWS_LAST = NPAD - ROWS_PT * (NS - 1)
TROWS_PT = 632
TROWS_LAST = N - TROWS_PT * (NS - 1)

_MESH = plsc.VectorSubcoreMesh(
    core_axis_name="c", subcore_axis_name="s", num_cores=NC, num_subcores=NS
)
_SC_PARAMS = pltpu.CompilerParams(use_tc_tiling_on_sc=False)


def _wid():
    return lax.axis_index("s") * NC + lax.axis_index("c")


def _load_chunks(ehbm, which, ev, w):
    start = pl.multiple_of(CPT * w, 8)
    pltpu.sync_copy(ehbm.at[which, pl.ds(start, CPT)], ev)


@functools.partial(
    pl.kernel,
    out_type=jax.ShapeDtypeStruct((NC, NPAD), jnp.float32),
    mesh=_MESH,
    scratch_types=[
        pltpu.VMEM((CPT, CHUNK), jnp.int32),
        pltpu.VMEM((CHUNK,), jnp.float32),
        pltpu.VMEM_SHARED((NPAD,), jnp.float32),
        pltpu.SemaphoreType.DMA,
    ],
    compiler_params=_SC_PARAMS,
)
def _sc_degree(edges_hbm, zeros_hbm, out_hbm, dst_v, ones_v, acc, sem):
    c = lax.axis_index("c")
    s = lax.axis_index("s")
    _load_chunks(edges_hbm, 1, dst_v, _wid())
    for i in range(CHUNK // 16):
        ones_v[pl.ds(i * 16, 16)] = jnp.ones((16,), jnp.float32)

    @pl.when(s == 0)
    def _():
        pltpu.sync_copy(zeros_hbm, acc)

    plsc.subcore_barrier()

    def fire(j, carry):
        pltpu.async_copy(ones_v, acc.at[dst_v.at[j]], sem, add=True)
        return carry

    lax.fori_loop(0, CPT, fire, 0)

    def drain(j, carry):
        pltpu.make_async_copy(ones_v, acc.at[dst_v.at[0]], sem).wait()
        return carry

    lax.fori_loop(0, CPT, drain, 0)
    plsc.subcore_barrier()

    @pl.when(s == 0)
    def _():
        pltpu.sync_copy(acc, out_hbm.at[c])


@functools.partial(
    pl.kernel,
    out_type=jax.ShapeDtypeStruct((NC, NPAD, HID), jnp.float32),
    mesh=_MESH,
    scratch_types=[
        pltpu.VMEM((CPT, CHUNK), jnp.int32),
        pltpu.VMEM((CPT, CHUNK), jnp.int32),
        [pltpu.VMEM((CHUNK, HID), jnp.float32)] * NBUF,
        [pltpu.SemaphoreType.DMA] * NBUF,
        [pltpu.SemaphoreType.DMA] * NBUF,
        pltpu.VMEM_SHARED((NPAD, HID), jnp.float32),
        pltpu.VMEM_SHARED((N, HID), jnp.float32),
    ],
    compiler_params=_SC_PARAMS,
)
def _sc_propagate(table_hbm, edges_hbm, zeros_hbm, out_hbm,
                  src_v, dst_v, rows, gsem, ssem, acc, table):
    c = lax.axis_index("c")
    s = lax.axis_index("s")
    w = _wid()
    _load_chunks(edges_hbm, 0, src_v, w)
    _load_chunks(edges_hbm, 1, dst_v, w)

    abase = pl.multiple_of(s * ROWS_PT, 8)
    tbase = pl.multiple_of(s * TROWS_PT, 8)

    @pl.when(s < NS - 1)
    def _():
        pltpu.sync_copy(zeros_hbm.at[pl.ds(abase, ROWS_PT)],
                        acc.at[pl.ds(abase, ROWS_PT)])
        pltpu.sync_copy(table_hbm.at[pl.ds(tbase, TROWS_PT)],
                        table.at[pl.ds(tbase, TROWS_PT)])

    @pl.when(s == NS - 1)
    def _():
        pltpu.sync_copy(
            zeros_hbm.at[pl.ds((NS - 1) * ROWS_PT, ROWS_LAST)],
            acc.at[pl.ds((NS - 1) * ROWS_PT, ROWS_LAST)])
        pltpu.sync_copy(
            table_hbm.at[pl.ds((NS - 1) * TROWS_PT, TROWS_LAST)],
            table.at[pl.ds((NS - 1) * TROWS_PT, TROWS_LAST)])

    plsc.subcore_barrier()

    def gather(k, i):
        pltpu.async_copy(table.at[src_v.at[k]], rows[i], gsem[i])

    def gather_done(i):
        pltpu.make_async_copy(table.at[src_v.at[0]], rows[i],
                              gsem[i]).wait()

    def scatter(k, i):
        pltpu.async_copy(rows[i], acc.at[dst_v.at[k]], ssem[i], add=True)

    def scatter_done(i):
        pltpu.make_async_copy(rows[i], acc.at[dst_v.at[0]], ssem[i]).wait()

    for i in range(NBUF):
        gather(i, i)

    def superchunk(t, carry):
        base = NBUF * t
        for i in range(NBUF):
            k = base + i
            gather_done(i)
            scatter(k, i)

            @pl.when(t < NSUP - 1)
            def _():
                scatter_done(i)
                gather(k + NBUF, i)
        return carry

    lax.fori_loop(0, NSUP, superchunk, 0)
    for i in range(NBUF):
        scatter_done(i)
    plsc.subcore_barrier()

    @pl.when(s < NS - 1)
    def _():
        pltpu.sync_copy(acc.at[pl.ds(abase, ROWS_PT)],
                        out_hbm.at[c, pl.ds(abase, ROWS_PT)])

    @pl.when(s == NS - 1)
    def _():
        pltpu.sync_copy(
            acc.at[pl.ds((NS - 1) * ROWS_PT, ROWS_LAST)],
            out_hbm.at[c, pl.ds((NS - 1) * ROWS_PT, ROWS_LAST)])


def _tc_mm_body(x_ref, w1_ref, xw_ref):
    xw_ref[...] = jnp.dot(x_ref[...], w1_ref[...],
                          preferred_element_type=jnp.float32)


def _tc1_body(xw_ref, degp_ref, xs1_ref, dis_ref):
    deg = sum(degp_ref[i, :N] for i in range(NC)) + 1.0
    dis = lax.rsqrt(deg)
    dis_ref[...] = dis
    xs1_ref[...] = xw_ref[...] * dis


def _tc2_body(y1p_ref, xs1_ref, dis_ref, b1_ref, xs2_ref):
    y = sum(y1p_ref[i, :N] for i in range(NC)) + xs1_ref[...]
    h = jnp.maximum(dis_ref[...] * y + b1_ref[...], 0.0)
    xs2_ref[...] = h * dis_ref[...]


def _tc3_body(y2p_ref, xs2_ref, dis_ref, w2_ref, b2_ref, out_ref):
    g = dis_ref[...] * (sum(y2p_ref[i, :N] for i in range(NC)) + xs2_ref[...])
    o = jnp.dot(g, w2_ref[...], preferred_element_type=jnp.float32) + b2_ref[...]
    m = jnp.max(o, axis=1, keepdims=True)
    e = o - m
    lse = jnp.log(jnp.sum(jnp.exp(e), axis=1, keepdims=True))
    out_ref[...] = e - lse


def kernel(x, edge_index, W1, b1, W2, b2):
    e32 = edge_index.astype(jnp.int32)
    padcols = jnp.stack([
        jnp.zeros((EPAD - E,), jnp.int32),
        jnp.full((EPAD - E,), DISCARD, jnp.int32),
    ])
    edges = jnp.concatenate([e32, padcols], axis=1).reshape(2, NCHUNK, CHUNK)
    z1 = jnp.zeros((NPAD,), jnp.float32)
    z16 = jnp.zeros((NPAD, HID), jnp.float32)

    degp = _sc_degree(edges, z1)
    xw = pl.pallas_call(
        _tc_mm_body,
        out_shape=jax.ShapeDtypeStruct((N, HID), jnp.float32),
    )(x, W1)

    xs1, dis = pl.pallas_call(
        _tc1_body,
        out_shape=(
            jax.ShapeDtypeStruct((N, HID), jnp.float32),
            jax.ShapeDtypeStruct((N, 1), jnp.float32),
        ),
    )(xw, degp.reshape(NC, NPAD, 1))

    y1p = _sc_propagate(xs1, edges, z16)

    xs2 = pl.pallas_call(
        _tc2_body,
        out_shape=jax.ShapeDtypeStruct((N, HID), jnp.float32),
    )(y1p, xs1, dis, b1.reshape(1, HID))

    y2p = _sc_propagate(xs2, edges, z16)

    out = pl.pallas_call(
        _tc3_body,
        out_shape=jax.ShapeDtypeStruct((N, OUT_CH), jnp.float32),
    )(y2p, xs2, dis, W2, b2.reshape(1, OUT_CH))
    return out
